# baseline (speedup 1.0000x reference)
import jax
import jax.numpy as jnp
from jax.experimental import pallas as pl
from jax.experimental.pallas import tpu as pltpu

NQ = 8
KC = 1024
CD = 32
DD = 256
TT = 2048

_HI = jax.lax.Precision.DEFAULT


def _dot(a, b):
    return jax.lax.dot_general(a, b, (((1,), (0,)), ((), ())),
                               preferred_element_type=jnp.float32,
                               precision=_HI)


def _vq_body(x_ref, inw_ref, inb_ref, outw_ref, outb_ref, cb2n_ref, c2_ref,
             cbt_ref, qout_ref, idx_ref, loss_ref):
    ress = [x_ref[0], x_ref[1]]
    accs = [jnp.zeros((DD, TT), jnp.float32) for _ in range(2)]
    for q in range(NQ):
        lparts = []
        for bb in range(2):
            z_e = _dot(inw_ref[q], ress[bb]) + inb_ref[q][:, None]
            nrm = jnp.sqrt(jnp.sum(z_e * z_e, axis=0, keepdims=True))
            z_en = z_e * (1.0 / jnp.maximum(nrm, 1e-12))
            dots2 = _dot(cb2n_ref[q], z_en)
            negd = dots2 - c2_ref[q]
            kio = jax.lax.broadcasted_iota(jnp.int32, (KC, TT), 0)
            idx = jnp.argmax(negd, axis=0)
            idx_ref[bb, q, :] = idx
            oh = (kio == idx[None, :]).astype(jnp.float32)
            z_q = _dot(cbt_ref[q], oh)
            diff = z_e - z_q
            lparts.append(jnp.sum(diff * diff))
            quant = _dot(outw_ref[q], z_q) + outb_ref[q][:, None]
            ress[bb] = ress[bb] - quant
            accs[bb] = accs[bb] + quant
        loss_ref[0, 0, q] = lparts[0] + lparts[1]
    qout_ref[0] = accs[0]
    qout_ref[1] = accs[1]


def kernel(x, params):
    B, D, T = x.shape
    inw = jnp.stack([p["in_w"] for p in params])
    inb = jnp.stack([p["in_b"] for p in params])
    outw = jnp.stack([p["out_w"] for p in params])
    outb = jnp.stack([p["out_b"] for p in params])
    cb = jnp.stack([p["codebook"] for p in params])
    cbt = jnp.transpose(cb, (0, 2, 1))
    cbn = jnp.linalg.norm(cb, axis=-1, keepdims=True)
    cb_n = cb / jnp.maximum(cbn, 1e-12)
    cb2n = 2.0 * cb_n
    c2 = jnp.sum(cb_n * cb_n, axis=2, keepdims=True)
    nt = T // TT

    qout, idx_bqt, loss_parts = pl.pallas_call(
        _vq_body,
        grid=(B // 2, nt),
        in_specs=[
            pl.BlockSpec((2, D, TT), lambda b, t: (b, 0, t)),
            pl.BlockSpec((NQ, CD, DD), lambda b, t: (0, 0, 0)),
            pl.BlockSpec((NQ, CD), lambda b, t: (0, 0)),
            pl.BlockSpec((NQ, DD, CD), lambda b, t: (0, 0, 0)),
            pl.BlockSpec((NQ, DD), lambda b, t: (0, 0)),
            pl.BlockSpec((NQ, KC, CD), lambda b, t: (0, 0, 0)),
            pl.BlockSpec((NQ, KC, 1), lambda b, t: (0, 0, 0)),
            pl.BlockSpec((NQ, CD, KC), lambda b, t: (0, 0, 0)),
        ],
        out_specs=[
            pl.BlockSpec((2, D, TT), lambda b, t: (b, 0, t)),
            pl.BlockSpec((2, NQ, TT), lambda b, t: (b, 0, t)),
            pl.BlockSpec((1, 1, NQ), lambda b, t: (b * nt + t, 0, 0),
                         memory_space=pltpu.SMEM),
        ],
        out_shape=[
            jax.ShapeDtypeStruct((B, D, T), jnp.float32),
            jax.ShapeDtypeStruct((B, NQ, T), jnp.int32),
            jax.ShapeDtypeStruct((B // 2 * nt, 1, NQ), jnp.float32),
        ],
        compiler_params=pltpu.CompilerParams(
            dimension_semantics=("parallel", "parallel")),
    )(x, inw, inb, outw, outb, cb2n, c2, cbt)

    indices = jnp.transpose(idx_bqt, (1, 0, 2))
    losses = 1.25 * jnp.sum(loss_parts, axis=(0, 1)) / (B * CD * T)
    return qout, indices, losses


# speedup vs baseline: 1.2922x; 1.2922x over previous
import jax
import jax.numpy as jnp
from jax.experimental import pallas as pl
from jax.experimental.pallas import tpu as pltpu

NQ = 8
KC = 1024
CD = 32
DD = 256
TT = 2048

_HI = jax.lax.Precision.DEFAULT


def _dot(a, b):
    return jax.lax.dot_general(a, b, (((1,), (0,)), ((), ())),
                               preferred_element_type=jnp.float32,
                               precision=_HI)


def _vq_body(x_ref, inw_ref, inb_ref, outw_ref, outb_ref, cb2n_ref, c2_ref,
             cbt_ref, qout_ref, idx_ref, loss_ref):
    res = x_ref[0]
    acc = jnp.zeros_like(res)
    for q in range(NQ):
        z_e = _dot(inw_ref[q], res) + inb_ref[q][:, None]
        nrm = jnp.sqrt(jnp.sum(z_e * z_e, axis=0, keepdims=True))
        z_en = z_e * (1.0 / jnp.maximum(nrm, 1e-12))
        dots2 = _dot(cb2n_ref[q], z_en)
        negd = dots2 - c2_ref[q]
        kio = jax.lax.broadcasted_iota(jnp.int32, (KC, TT), 0)
        idx = jnp.argmax(negd, axis=0)
        idx_ref[0, q, :] = idx
        oh = (kio == idx[None, :]).astype(jnp.float32)
        z_q = _dot(cbt_ref[q], oh)
        diff = z_e - z_q
        loss_ref[0, 0, q] = jnp.sum(diff * diff)
        quant = _dot(outw_ref[q], z_q) + outb_ref[q][:, None]
        res = res - quant
        acc = acc + quant
    qout_ref[0] = acc


def kernel(x, params):
    B, D, T = x.shape
    inw = jnp.stack([p["in_w"] for p in params])
    inb = jnp.stack([p["in_b"] for p in params])
    outw = jnp.stack([p["out_w"] for p in params])
    outb = jnp.stack([p["out_b"] for p in params])
    cb = jnp.stack([p["codebook"] for p in params])
    cbt = jnp.transpose(cb, (0, 2, 1))
    cbn = jnp.linalg.norm(cb, axis=-1, keepdims=True)
    cb_n = cb / jnp.maximum(cbn, 1e-12)
    cb2n = 2.0 * cb_n
    c2 = jnp.sum(cb_n * cb_n, axis=2, keepdims=True)
    nt = T // TT

    qout, idx_bqt, loss_parts = pl.pallas_call(
        _vq_body,
        grid=(B, nt),
        in_specs=[
            pl.BlockSpec((1, D, TT), lambda b, t: (b, 0, t)),
            pl.BlockSpec((NQ, CD, DD), lambda b, t: (0, 0, 0)),
            pl.BlockSpec((NQ, CD), lambda b, t: (0, 0)),
            pl.BlockSpec((NQ, DD, CD), lambda b, t: (0, 0, 0)),
            pl.BlockSpec((NQ, DD), lambda b, t: (0, 0)),
            pl.BlockSpec((NQ, KC, CD), lambda b, t: (0, 0, 0)),
            pl.BlockSpec((NQ, KC, 1), lambda b, t: (0, 0, 0)),
            pl.BlockSpec((NQ, CD, KC), lambda b, t: (0, 0, 0)),
        ],
        out_specs=[
            pl.BlockSpec((1, D, TT), lambda b, t: (b, 0, t)),
            pl.BlockSpec((1, NQ, TT), lambda b, t: (b, 0, t)),
            pl.BlockSpec((1, 1, NQ), lambda b, t: (b * nt + t, 0, 0),
                         memory_space=pltpu.SMEM),
        ],
        out_shape=[
            jax.ShapeDtypeStruct((B, D, T), jnp.float32),
            jax.ShapeDtypeStruct((B, NQ, T), jnp.int32),
            jax.ShapeDtypeStruct((B * nt, 1, NQ), jnp.float32),
        ],
        compiler_params=pltpu.CompilerParams(
            dimension_semantics=("parallel", "parallel")),
    )(x, inw, inb, outw, outb, cb2n, c2, cbt)

    indices = jnp.transpose(idx_bqt, (1, 0, 2))
    losses = 1.25 * jnp.sum(loss_parts, axis=(0, 1)) / (B * CD * T)
    return qout, indices, losses
